# baseline (speedup 1.0000x reference)
import functools
import math

import jax
import jax.numpy as jnp
from jax import lax
from jax.experimental import pallas as pl
from jax.experimental.pallas import tpu as pltpu
from jax.experimental.pallas import tpu_sc as plsc

_TOPK = 128
_SEQ = 4096
_DK = 64
_INT_MIN = -2147483648
_DUMP = 144


def _importance_body(q_ref, ms_ref, qp_ref):
    xt = q_ref[0]
    mean = jnp.mean(xt, axis=0)
    xc = xt - mean[None, :]
    var = jnp.sum(xc * xc, axis=0) * (1.0 / (_DK - 1))
    imp = mean + jnp.sqrt(var)

    u = lax.bitcast_convert_type(imp, jnp.int32)
    ms = jnp.where(u >= 0, u, u ^ jnp.int32(0x7FFFFFFF))
    ms_ref[0] = ms.reshape(_SEQ // 128, 128)
    eye = (lax.broadcasted_iota(jnp.int32, (_DK, _DK), 0)
           == lax.broadcasted_iota(jnp.int32, (_DK, _DK), 1)
           ).astype(jnp.float32)
    x = lax.dot_general(xt, eye, (((0,), (0,)), ((), ())),
                        preferred_element_type=jnp.float32)
    x3 = x.reshape(_SEQ // 2, 2, _DK)
    qp_ref[0] = jnp.concatenate([x3[:, 0, :], x3[:, 1, :]], axis=1)


def _run_importance(qt3):
    n = qt3.shape[0]
    return pl.pallas_call(
        _importance_body,
        grid=(n,),
        in_specs=[pl.BlockSpec((1, _DK, _SEQ), lambda i: (i, 0, 0))],
        out_specs=[
            pl.BlockSpec((1, _SEQ // 128, 128), lambda i: (i, 0, 0)),
            pl.BlockSpec((1, _SEQ // 2, 2 * _DK), lambda i: (i, 0, 0)),
        ],
        out_shape=[
            jax.ShapeDtypeStruct((n, _SEQ // 128, 128), jnp.int32),
            jax.ShapeDtypeStruct((n, _SEQ // 2, 2 * _DK), jnp.float32),
        ],
    )(qt3)


def _rank_body(ms_ref, lidx_ref, gp_ref, dest_scr):
    nr = ms_ref.shape[0]
    nc = _SEQ // 128
    m3 = ms_ref[...]
    m2d = m3.reshape(nr, _SEQ)

    cnt0 = jnp.sum((m2d >= 0).astype(jnp.int32), axis=1, keepdims=True)
    prefix0 = jnp.where(cnt0 >= _TOPK, jnp.int32(0), jnp.int32(_INT_MIN))

    def bit_body(b, prefix):
        bit = lax.shift_left(jnp.int32(1), jnp.int32(30) - b)
        cand = prefix | bit
        cnt = jnp.sum((m2d >= cand).astype(jnp.int32), axis=1, keepdims=True)
        return jnp.where(cnt >= _TOPK, cand, prefix)

    t = lax.fori_loop(0, 31, bit_body, prefix0)
    t3 = t[:, :, None]

    gt = m3 > t3
    eq = m3 == t3
    gtf = gt.astype(jnp.float32)
    eqf = eq.astype(jnp.float32)

    u128 = (lax.broadcasted_iota(jnp.int32, (128, 128), 0)
            < lax.broadcasted_iota(jnp.int32, (128, 128), 1)).astype(jnp.float32)
    u32s = (lax.broadcasted_iota(jnp.int32, (nc, nc), 0)
            < lax.broadcasted_iota(jnp.int32, (nc, nc), 1)).astype(jnp.float32)

    def ex_prefix(f3):
        lane = lax.dot_general(f3.reshape(nr * nc, 128), u128,
                               (((1,), (0,)), ((), ())),
                               preferred_element_type=jnp.float32)
        chs = jnp.sum(f3, axis=2)
        chpre = lax.dot_general(chs, u32s, (((1,), (0,)), ((), ())),
                                preferred_element_type=jnp.float32)
        return lane.reshape(nr, nc, 128) + chpre[:, :, None]

    c_gt = jnp.sum(jnp.sum(gtf, axis=2), axis=1)[:, None, None]
    r = jnp.float32(_TOPK) - c_gt
    peq = ex_prefix(eqf)
    sel = gt | (eq & (peq < r))
    psel = ex_prefix(sel.astype(jnp.float32))
    dest = jnp.where(sel, psel, jnp.float32(_TOPK)).astype(jnp.int32)
    dest_scr[...] = dest

    tio = lax.broadcasted_iota(jnp.int32, (nc, 128, _TOPK), 2)
    flatf = (lax.broadcasted_iota(jnp.int32, (nc, 128), 0) * 128
             + lax.broadcasted_iota(jnp.int32, (nc, 128), 1))

    def row_body(i, carry):
        d2 = dest_scr[pl.ds(i, 1)][0]
        contrib = jnp.where(d2[:, :, None] == tio, flatf[:, :, None], 0)
        inv = jnp.sum(jnp.sum(contrib, axis=0), axis=0)
        lidx_ref[pl.ds(i, 1), :] = inv.reshape(1, _TOPK)
        gp_ref[pl.ds(i, 1), :] = (
            (inv + jnp.int32(_SEQ) * i) >> 1).reshape(1, _TOPK)
        return carry

    lax.fori_loop(0, nr, row_body, jnp.int32(0))


def _run_rank(ms):
    n = ms.shape[0]
    return pl.pallas_call(
        _rank_body,
        out_shape=[
            jax.ShapeDtypeStruct((n, _TOPK), jnp.int32),
            jax.ShapeDtypeStruct((n, _TOPK), jnp.int32),
        ],
        scratch_shapes=[pltpu.VMEM((n, _SEQ // 128, 128), jnp.int32)],
    )(ms)


def _sc_body(gidx_hbm, q_hbm, qs_hbm, gidx_v, qs_v, sem):
    p = lax.axis_index("s") * 2 + lax.axis_index("c")

    pltpu.sync_copy(gidx_hbm.at[p], gidx_v)

    pltpu.async_copy(q_hbm.at[gidx_v], qs_v, sem).wait()
    pltpu.sync_copy(qs_v, qs_hbm.at[p])


def _run_select_gather(gidx2, q2):
    n = gidx2.shape[0]
    mesh = plsc.VectorSubcoreMesh(core_axis_name="c", subcore_axis_name="s")
    f = functools.partial(
        pl.kernel,
        mesh=mesh,
        out_type=jax.ShapeDtypeStruct((n, _TOPK, 2 * _DK), jnp.float32),
        scratch_types=[
            pltpu.VMEM((_TOPK,), jnp.int32),
            pltpu.VMEM((_TOPK, 2 * _DK), jnp.float32),
            pltpu.SemaphoreType.DMA,
        ],
    )(_sc_body)
    return f(gidx2, q2)


def _attention_body(qs_ref, kt_ref, vt_ref, idx_ref, out_ref):
    local = idx_ref[0]
    par = (local & 1)[0][:, None] == 1

    qfull = qs_ref[0]
    qb = jnp.where(par, qfull[:, _DK:], qfull[:, :_DK])

    p = (lax.broadcasted_iota(jnp.int32, (_SEQ, _TOPK), 0)
         == local).astype(jnp.bfloat16)

    kvt = jnp.concatenate([kt_ref[0], vt_ref[0]], axis=0)
    hi = kvt.astype(jnp.bfloat16)
    lo = (kvt - hi.astype(jnp.float32)).astype(jnp.bfloat16)
    kvg = (lax.dot_general(hi, p, (((1,), (0,)), ((), ())),
                           preferred_element_type=jnp.float32)
           + lax.dot_general(lo, p, (((1,), (0,)), ((), ())),
                             preferred_element_type=jnp.float32))
    ktg = kvg[:_DK]
    vtg = kvg[_DK:]
    s = lax.dot_general(qb, ktg, (((1,), (0,)), ((), ())),
                        preferred_element_type=jnp.float32)
    s = s * (1.0 / math.sqrt(_DK))
    mx = jnp.max(s, axis=-1, keepdims=True)
    e = jnp.exp(s - mx)
    w = e / jnp.sum(e, axis=-1, keepdims=True)
    att = lax.dot_general(w, vtg, (((1,), (1,)), ((), ())),
                          preferred_element_type=jnp.float32)
    a_hi = att.astype(jnp.bfloat16)
    a_lo = (att - a_hi.astype(jnp.float32)).astype(jnp.bfloat16)
    out = (lax.dot_general(p, a_hi, (((1,), (0,)), ((), ())),
                           preferred_element_type=jnp.float32)
           + lax.dot_general(p, a_lo, (((1,), (0,)), ((), ())),
                             preferred_element_type=jnp.float32))
    out_ref[0] = out.reshape(_SEQ // _DK, _DK, _DK)


def _run_attention(qs, kt3, vt3, idx3, batch):
    n = qs.shape[0]
    hpb = n // batch
    return pl.pallas_call(
        _attention_body,
        grid=(n,),
        in_specs=[
            pl.BlockSpec((1, _TOPK, 2 * _DK), lambda i: (i, 0, 0)),
            pl.BlockSpec((1, _DK, _SEQ), lambda i: (i, 0, 0)),
            pl.BlockSpec((1, _DK, _SEQ), lambda i: (i, 0, 0)),
            pl.BlockSpec((1, 1, _TOPK), lambda i: (i, 0, 0)),
        ],
        out_specs=pl.BlockSpec(
            (1, _SEQ // _DK, _DK, _DK),
            lambda i: (i // hpb, i % hpb, 0, 0)),
        out_shape=jax.ShapeDtypeStruct(
            (batch, hpb * (_SEQ // _DK), _DK, _DK), jnp.float32),
    )(qs, kt3, vt3, idx3)


def kernel(q, k, v):
    B, H, S, D = q.shape
    n = B * H
    qt3 = jnp.transpose(q, (0, 1, 3, 2)).reshape(n, D, S)
    kt3 = jnp.transpose(k, (0, 1, 3, 2)).reshape(n, D, S)
    vt3 = jnp.transpose(v, (0, 1, 3, 2)).reshape(n, D, S)

    ms, qpack = _run_importance(qt3)
    lidx, gpair = _run_rank(ms)

    qs = _run_select_gather(gpair, qpack.reshape(n * S // 2, 2 * D))

    return _run_attention(qs, kt3, vt3, lidx.reshape(n, 1, _TOPK), B)


# speedup vs baseline: 1.0101x; 1.0101x over previous
import functools
import math

import jax
import jax.numpy as jnp
from jax import lax
from jax.experimental import pallas as pl
from jax.experimental.pallas import tpu as pltpu
from jax.experimental.pallas import tpu_sc as plsc

_TOPK = 128
_SEQ = 4096
_DK = 64
_INT_MIN = -2147483648
_DUMP = 144


def _importance_body(q_ref, ms_ref, qp_ref):
    xt = q_ref[0]
    mean = jnp.mean(xt, axis=0)
    xc = xt - mean[None, :]
    var = jnp.sum(xc * xc, axis=0) * (1.0 / (_DK - 1))
    imp = mean + jnp.sqrt(var)

    u = lax.bitcast_convert_type(imp, jnp.int32)
    ms = jnp.where(u >= 0, u, u ^ jnp.int32(0x7FFFFFFF))
    ms_ref[0] = ms.reshape(_SEQ // 128, 128)
    eye = (lax.broadcasted_iota(jnp.int32, (_DK, _DK), 0)
           == lax.broadcasted_iota(jnp.int32, (_DK, _DK), 1)
           ).astype(jnp.float32)
    x = lax.dot_general(xt, eye, (((0,), (0,)), ((), ())),
                        preferred_element_type=jnp.float32)
    x3 = x.reshape(_SEQ // 2, 2, _DK)
    qp_ref[0] = jnp.concatenate([x3[:, 0, :], x3[:, 1, :]], axis=1)


def _run_importance(qt3):
    n = qt3.shape[0]
    return pl.pallas_call(
        _importance_body,
        grid=(n,),
        in_specs=[pl.BlockSpec((1, _DK, _SEQ), lambda i: (i, 0, 0))],
        out_specs=[
            pl.BlockSpec((1, _SEQ // 128, 128), lambda i: (i, 0, 0)),
            pl.BlockSpec((1, _SEQ // 2, 2 * _DK), lambda i: (i, 0, 0)),
        ],
        out_shape=[
            jax.ShapeDtypeStruct((n, _SEQ // 128, 128), jnp.int32),
            jax.ShapeDtypeStruct((n, _SEQ // 2, 2 * _DK), jnp.float32),
        ],
    )(qt3)


def _rank_body(ms_ref, lidx_ref, gp_ref, dest_scr):
    nr = ms_ref.shape[0]
    nc = _SEQ // 128
    m3 = ms_ref[...]
    m2d = m3.reshape(nr, _SEQ)

    cnt0 = jnp.sum((m2d >= 0).astype(jnp.int32), axis=1, keepdims=True)
    prefix0 = jnp.where(cnt0 >= _TOPK, jnp.int32(0), jnp.int32(_INT_MIN))

    def bit_body(b, prefix):
        bit = lax.shift_left(jnp.int32(1), jnp.int32(30) - b)
        cand = prefix | bit
        cnt = jnp.sum((m2d >= cand).astype(jnp.int32), axis=1, keepdims=True)
        return jnp.where(cnt >= _TOPK, cand, prefix)

    t = lax.fori_loop(0, 31, bit_body, prefix0)
    t3 = t[:, :, None]

    gt = m3 > t3
    eq = m3 == t3
    gtf = gt.astype(jnp.float32)
    eqf = eq.astype(jnp.float32)

    u128 = (lax.broadcasted_iota(jnp.int32, (128, 128), 0)
            < lax.broadcasted_iota(jnp.int32, (128, 128), 1)).astype(jnp.float32)
    u32s = (lax.broadcasted_iota(jnp.int32, (nc, nc), 0)
            < lax.broadcasted_iota(jnp.int32, (nc, nc), 1)).astype(jnp.float32)

    def ex_prefix(f3):
        lane = lax.dot_general(f3.reshape(nr * nc, 128), u128,
                               (((1,), (0,)), ((), ())),
                               preferred_element_type=jnp.float32)
        chs = jnp.sum(f3, axis=2)
        chpre = lax.dot_general(chs, u32s, (((1,), (0,)), ((), ())),
                                preferred_element_type=jnp.float32)
        return lane.reshape(nr, nc, 128) + chpre[:, :, None]

    c_gt = jnp.sum(jnp.sum(gtf, axis=2), axis=1)[:, None, None]
    r = jnp.float32(_TOPK) - c_gt
    peq = ex_prefix(eqf)
    sel = gt | (eq & (peq < r))
    psel = ex_prefix(sel.astype(jnp.float32))
    dest = jnp.where(sel, psel, jnp.float32(_TOPK)).astype(jnp.int32)
    dest_scr[...] = dest

    tio = lax.broadcasted_iota(jnp.int32, (nc, 128, _TOPK), 2)
    flatf = (lax.broadcasted_iota(jnp.int32, (nc, 128), 0) * 128
             + lax.broadcasted_iota(jnp.int32, (nc, 128), 1))

    def row_body(i, carry):
        d2 = dest_scr[pl.ds(i, 1)][0]
        contrib = jnp.where(d2[:, :, None] == tio, flatf[:, :, None], 0)
        inv = jnp.sum(jnp.sum(contrib, axis=0), axis=0)
        lidx_ref[pl.ds(i, 1), :] = inv.reshape(1, _TOPK)
        gp_ref[pl.ds(i, 1), :] = (
            (inv + jnp.int32(_SEQ) * i) >> 1).reshape(1, _TOPK)
        return carry

    lax.fori_loop(0, nr, row_body, jnp.int32(0))


def _run_rank(ms):
    n = ms.shape[0]
    return pl.pallas_call(
        _rank_body,
        out_shape=[
            jax.ShapeDtypeStruct((n, _TOPK), jnp.int32),
            jax.ShapeDtypeStruct((n, _TOPK), jnp.int32),
        ],
        scratch_shapes=[pltpu.VMEM((n, _SEQ // 128, 128), jnp.int32)],
    )(ms)


def _sc_body(gidx_hbm, q_hbm, qs_hbm, gidx_v, qs_v, sem):
    p = lax.axis_index("s") * 2 + lax.axis_index("c")

    pltpu.sync_copy(gidx_hbm.at[p], gidx_v)

    pltpu.async_copy(q_hbm.at[gidx_v], qs_v, sem).wait()
    pltpu.sync_copy(qs_v, qs_hbm.at[p])


def _run_select_gather(gidx2, q2):
    n = gidx2.shape[0]
    mesh = plsc.VectorSubcoreMesh(core_axis_name="c", subcore_axis_name="s")
    f = functools.partial(
        pl.kernel,
        mesh=mesh,
        out_type=jax.ShapeDtypeStruct((n, _TOPK, 2 * _DK), jnp.float32),
        scratch_types=[
            pltpu.VMEM((_TOPK,), jnp.int32),
            pltpu.VMEM((_TOPK, 2 * _DK), jnp.float32),
            pltpu.SemaphoreType.DMA,
        ],
    )(_sc_body)
    return f(gidx2, q2)


def _attention_body(qs_ref, kt_ref, vt_ref, idx_ref, out_ref):
    local = idx_ref[0]
    par = (local & 1)[0][:, None] == 1

    qfull = qs_ref[0]
    qb = jnp.where(par, qfull[:, _DK:], qfull[:, :_DK])

    p = (lax.broadcasted_iota(jnp.int32, (_SEQ, _TOPK), 0)
         == local).astype(jnp.bfloat16)

    def gather_cols(ref):
        full = ref[0]
        hi = full.astype(jnp.bfloat16)
        lo = (full - hi.astype(jnp.float32)).astype(jnp.bfloat16)
        return (lax.dot_general(hi, p, (((1,), (0,)), ((), ())),
                                preferred_element_type=jnp.float32)
                + lax.dot_general(lo, p, (((1,), (0,)), ((), ())),
                                  preferred_element_type=jnp.float32))

    ktg = gather_cols(kt_ref)
    vtg = gather_cols(vt_ref)
    s = lax.dot_general(qb, ktg, (((1,), (0,)), ((), ())),
                        preferred_element_type=jnp.float32)
    s = s * (1.0 / math.sqrt(_DK))
    mx = jnp.max(s, axis=-1, keepdims=True)
    e = jnp.exp(s - mx)
    w = e / jnp.sum(e, axis=-1, keepdims=True)
    att = lax.dot_general(w, vtg, (((1,), (1,)), ((), ())),
                          preferred_element_type=jnp.float32)
    a_hi = att.astype(jnp.bfloat16)
    a_lo = (att - a_hi.astype(jnp.float32)).astype(jnp.bfloat16)
    out = (lax.dot_general(p, a_hi, (((1,), (0,)), ((), ())),
                           preferred_element_type=jnp.float32)
           + lax.dot_general(p, a_lo, (((1,), (0,)), ((), ())),
                             preferred_element_type=jnp.float32))
    out_ref[0] = out.reshape(_SEQ // _DK, _DK, _DK)


def _run_attention(qs, kt3, vt3, idx3, batch):
    n = qs.shape[0]
    hpb = n // batch
    return pl.pallas_call(
        _attention_body,
        grid=(n,),
        in_specs=[
            pl.BlockSpec((1, _TOPK, 2 * _DK), lambda i: (i, 0, 0)),
            pl.BlockSpec((1, _DK, _SEQ), lambda i: (i, 0, 0)),
            pl.BlockSpec((1, _DK, _SEQ), lambda i: (i, 0, 0)),
            pl.BlockSpec((1, 1, _TOPK), lambda i: (i, 0, 0)),
        ],
        out_specs=pl.BlockSpec(
            (1, _SEQ // _DK, _DK, _DK),
            lambda i: (i // hpb, i % hpb, 0, 0)),
        out_shape=jax.ShapeDtypeStruct(
            (batch, hpb * (_SEQ // _DK), _DK, _DK), jnp.float32),
    )(qs, kt3, vt3, idx3)


def kernel(q, k, v):
    B, H, S, D = q.shape
    n = B * H
    qt3 = jnp.transpose(q, (0, 1, 3, 2)).reshape(n, D, S)
    kt3 = jnp.transpose(k, (0, 1, 3, 2)).reshape(n, D, S)
    vt3 = jnp.transpose(v, (0, 1, 3, 2)).reshape(n, D, S)

    ms, qpack = _run_importance(qt3)
    lidx, gpair = _run_rank(ms)

    qs = _run_select_gather(gpair, qpack.reshape(n * S // 2, 2 * D))

    return _run_attention(qs, kt3, vt3, lidx.reshape(n, 1, _TOPK), B)


# speedup vs baseline: 1.0170x; 1.0068x over previous
import functools
import math

import jax
import jax.numpy as jnp
from jax import lax
from jax.experimental import pallas as pl
from jax.experimental.pallas import tpu as pltpu
from jax.experimental.pallas import tpu_sc as plsc

_TOPK = 128
_SEQ = 4096
_DK = 64
_INT_MIN = -2147483648


def _importance_body(q_ref, ms_ref, qp_ref):
    xt = q_ref[0]
    mean = jnp.mean(xt, axis=0)
    xc = xt - mean[None, :]
    var = jnp.sum(xc * xc, axis=0) * (1.0 / (_DK - 1))
    imp = mean + jnp.sqrt(var)

    u = lax.bitcast_convert_type(imp, jnp.int32)
    ms = jnp.where(u >= 0, u, u ^ jnp.int32(0x7FFFFFFF))
    ms_ref[0] = ms.reshape(_SEQ // 128, 128)
    eye = (lax.broadcasted_iota(jnp.int32, (_DK, _DK), 0)
           == lax.broadcasted_iota(jnp.int32, (_DK, _DK), 1)
           ).astype(jnp.float32)
    x = lax.dot_general(xt, eye, (((0,), (0,)), ((), ())),
                        preferred_element_type=jnp.float32)
    x3 = x.reshape(_SEQ // 2, 2, _DK)
    qp_ref[0] = jnp.concatenate([x3[:, 0, :], x3[:, 1, :]], axis=1)


def _run_importance(qt3):
    n = qt3.shape[0]
    return pl.pallas_call(
        _importance_body,
        grid=(n,),
        in_specs=[pl.BlockSpec((1, _DK, _SEQ), lambda i: (i, 0, 0))],
        out_specs=[
            pl.BlockSpec((1, _SEQ // 128, 128), lambda i: (i, 0, 0)),
            pl.BlockSpec((1, _SEQ // 2, 2 * _DK), lambda i: (i, 0, 0)),
        ],
        out_shape=[
            jax.ShapeDtypeStruct((n, _SEQ // 128, 128), jnp.int32),
            jax.ShapeDtypeStruct((n, _SEQ // 2, 2 * _DK), jnp.float32),
        ],
    )(qt3)


def _rank_body(ms_ref, lidx_ref, gp_ref, dest_scr):
    nr = ms_ref.shape[0]
    nc = _SEQ // 128
    m3 = ms_ref[...]
    m2d = m3.reshape(nr, _SEQ)

    cnt0 = jnp.sum((m2d >= 0).astype(jnp.int32), axis=1, keepdims=True)
    prefix0 = jnp.where(cnt0 >= _TOPK, jnp.int32(0), jnp.int32(_INT_MIN))

    def bit_body(b, prefix):
        bit = lax.shift_left(jnp.int32(1), jnp.int32(30) - b)
        cand = prefix | bit
        cnt = jnp.sum((m2d >= cand).astype(jnp.int32), axis=1, keepdims=True)
        return jnp.where(cnt >= _TOPK, cand, prefix)

    t = lax.fori_loop(0, 31, bit_body, prefix0)
    t3 = t[:, :, None]

    gt = m3 > t3
    eq = m3 == t3
    gtf = gt.astype(jnp.float32)
    eqf = eq.astype(jnp.float32)

    u128 = (lax.broadcasted_iota(jnp.int32, (128, 128), 0)
            < lax.broadcasted_iota(jnp.int32, (128, 128), 1)).astype(jnp.float32)
    u32s = (lax.broadcasted_iota(jnp.int32, (nc, nc), 0)
            < lax.broadcasted_iota(jnp.int32, (nc, nc), 1)).astype(jnp.float32)

    def ex_prefix(f3):
        lane = lax.dot_general(f3.reshape(nr * nc, 128), u128,
                               (((1,), (0,)), ((), ())),
                               preferred_element_type=jnp.float32)
        chs = jnp.sum(f3, axis=2)
        chpre = lax.dot_general(chs, u32s, (((1,), (0,)), ((), ())),
                                preferred_element_type=jnp.float32)
        return lane.reshape(nr, nc, 128) + chpre[:, :, None]

    c_gt = jnp.sum(jnp.sum(gtf, axis=2), axis=1)[:, None, None]
    r = jnp.float32(_TOPK) - c_gt
    peq = ex_prefix(eqf)
    sel = gt | (eq & (peq < r))
    psel = ex_prefix(sel.astype(jnp.float32))
    dest = jnp.where(sel, psel, jnp.float32(_TOPK)).astype(jnp.int32)
    dest_scr[...] = dest

    tio = lax.broadcasted_iota(jnp.int32, (nc, 128, _TOPK), 2)
    flatf = (lax.broadcasted_iota(jnp.int32, (nc, 128), 0) * 128
             + lax.broadcasted_iota(jnp.int32, (nc, 128), 1))

    def row_body(i, carry):
        d2 = dest_scr[pl.ds(i, 1)][0]
        e2 = (d2[:, :, None] == tio).astype(jnp.int32)
        contrib = e2 * flatf[:, :, None]
        inv = jnp.sum(jnp.sum(contrib, axis=0), axis=0)
        lidx_ref[pl.ds(i, 1), :] = inv.reshape(1, _TOPK)
        gp_ref[pl.ds(i, 1), :] = (
            (inv + jnp.int32(_SEQ) * i) >> 1).reshape(1, _TOPK)
        return carry

    lax.fori_loop(0, nr, row_body, jnp.int32(0))


def _run_rank(ms):
    n = ms.shape[0]
    return pl.pallas_call(
        _rank_body,
        out_shape=[
            jax.ShapeDtypeStruct((n, _TOPK), jnp.int32),
            jax.ShapeDtypeStruct((n, _TOPK), jnp.int32),
        ],
        scratch_shapes=[pltpu.VMEM((n, _SEQ // 128, 128), jnp.int32)],
    )(ms)


def _sc_body(gidx_hbm, q_hbm, qs_hbm, gidx_v, qs_v, sem):
    p = lax.axis_index("s") * 2 + lax.axis_index("c")

    pltpu.sync_copy(gidx_hbm.at[p], gidx_v)

    pltpu.async_copy(q_hbm.at[gidx_v], qs_v, sem).wait()
    pltpu.sync_copy(qs_v, qs_hbm.at[p])


def _run_select_gather(gidx2, q2):
    n = gidx2.shape[0]
    mesh = plsc.VectorSubcoreMesh(core_axis_name="c", subcore_axis_name="s")
    f = functools.partial(
        pl.kernel,
        mesh=mesh,
        out_type=jax.ShapeDtypeStruct((n, _TOPK, 2 * _DK), jnp.float32),
        scratch_types=[
            pltpu.VMEM((_TOPK,), jnp.int32),
            pltpu.VMEM((_TOPK, 2 * _DK), jnp.float32),
            pltpu.SemaphoreType.DMA,
        ],
    )(_sc_body)
    return f(gidx2, q2)


def _attention_body(qs_ref, kt_ref, vt_ref, idx_ref, out_ref):
    local = idx_ref[0]
    par = (local & 1)[0][:, None] == 1

    qfull = qs_ref[0]
    qb = jnp.where(par, qfull[:, _DK:], qfull[:, :_DK])

    p = (lax.broadcasted_iota(jnp.int32, (_SEQ, _TOPK), 0)
         == local).astype(jnp.bfloat16)

    def gather_cols(ref):
        full = ref[0]
        hi = full.astype(jnp.bfloat16)
        lo = (full - hi.astype(jnp.float32)).astype(jnp.bfloat16)
        return (lax.dot_general(hi, p, (((1,), (0,)), ((), ())),
                                preferred_element_type=jnp.float32)
                + lax.dot_general(lo, p, (((1,), (0,)), ((), ())),
                                  preferred_element_type=jnp.float32))

    ktg = gather_cols(kt_ref)
    vtg = gather_cols(vt_ref)
    s = lax.dot_general(qb, ktg, (((1,), (0,)), ((), ())),
                        preferred_element_type=jnp.float32)
    s = s * (1.0 / math.sqrt(_DK))
    mx = jnp.max(s, axis=-1, keepdims=True)
    e = jnp.exp(s - mx)
    w = e / jnp.sum(e, axis=-1, keepdims=True)
    att = lax.dot_general(w, vtg, (((1,), (1,)), ((), ())),
                          preferred_element_type=jnp.float32)
    a_hi = att.astype(jnp.bfloat16)
    a_lo = (att - a_hi.astype(jnp.float32)).astype(jnp.bfloat16)
    out = (lax.dot_general(p, a_hi, (((1,), (0,)), ((), ())),
                           preferred_element_type=jnp.float32)
           + lax.dot_general(p, a_lo, (((1,), (0,)), ((), ())),
                             preferred_element_type=jnp.float32))
    out_ref[0] = out.reshape(_SEQ // _DK, _DK, _DK)


def _run_attention(qs, kt3, vt3, idx3, batch):
    n = qs.shape[0]
    hpb = n // batch
    return pl.pallas_call(
        _attention_body,
        grid=(n,),
        in_specs=[
            pl.BlockSpec((1, _TOPK, 2 * _DK), lambda i: (i, 0, 0)),
            pl.BlockSpec((1, _DK, _SEQ), lambda i: (i, 0, 0)),
            pl.BlockSpec((1, _DK, _SEQ), lambda i: (i, 0, 0)),
            pl.BlockSpec((1, 1, _TOPK), lambda i: (i, 0, 0)),
        ],
        out_specs=pl.BlockSpec(
            (1, _SEQ // _DK, _DK, _DK),
            lambda i: (i // hpb, i % hpb, 0, 0)),
        out_shape=jax.ShapeDtypeStruct(
            (batch, hpb * (_SEQ // _DK), _DK, _DK), jnp.float32),
    )(qs, kt3, vt3, idx3)


def kernel(q, k, v):
    B, H, S, D = q.shape
    n = B * H
    qt3 = jnp.transpose(q, (0, 1, 3, 2)).reshape(n, D, S)
    kt3 = jnp.transpose(k, (0, 1, 3, 2)).reshape(n, D, S)
    vt3 = jnp.transpose(v, (0, 1, 3, 2)).reshape(n, D, S)

    ms, qpack = _run_importance(qt3)
    lidx, gpair = _run_rank(ms)

    qs = _run_select_gather(gpair, qpack.reshape(n * S // 2, 2 * D))

    return _run_attention(qs, kt3, vt3, lidx.reshape(n, 1, _TOPK), B)
